# baseline (speedup 1.0000x reference)
import jax
import jax.numpy as jnp
from jax import lax
from jax.experimental import pallas as pl
from jax.experimental.pallas import tpu as pltpu
from jax.experimental.pallas import tpu_sc as plsc

f32 = jnp.float32
i32 = jnp.int32

H = 128
N = 10000
E = 320000
B = 64
TMAX = 3200
NC, NS = 2, 16
NW = NC * NS
CH = 128

EPAD = NW * CH * 80
UPAD = NW * CH * 4
NA0 = 10240
NA1 = 3328


def _sc_scatter_sum(table, sd4, n_acc):
    nchunk = sd4.shape[1]
    assert nchunk % 2 == 0
    rpt = n_acc // NS
    mesh = plsc.VectorSubcoreMesh(core_axis_name="c", subcore_axis_name="s")

    def body(tab_ref, sd_ref, out_ref, acc, idx0, idx1, rows0, rows1,
             sg0, sg1):
        c = lax.axis_index("c")
        s = lax.axis_index("s")
        w = c * NS + s
        with jax.named_scope("agg_zero"):
            pltpu.sync_copy(sd_ref.at[w, 0], idx0)
            pltpu.async_copy(tab_ref.at[idx0.at[0]], rows0, sg0)
            zeros16 = jnp.zeros((16,), f32)

            def zrow(r, carry):
                for j in range(8):
                    rows1[r, pl.ds(j * 16, 16)] = zeros16
                return carry
            lax.fori_loop(0, CH, zrow, 0)
            for k in range(rpt // CH):
                pltpu.sync_copy(rows1, acc.at[pl.ds(s * rpt + k * CH, CH)])
            rem = rpt % CH
            if rem:
                pltpu.sync_copy(rows1.at[pl.ds(0, rem)],
                                acc.at[pl.ds(s * rpt + (rpt // CH) * CH, rem)])
            plsc.subcore_barrier()

        idx = (idx0, idx1)
        rows = (rows0, rows1)
        sg = (sg0, sg1)

        def halfstep(i, p):
            q = 1 - p
            nxt = lax.rem(i + 1, nchunk)
            pltpu.sync_copy(sd_ref.at[w, nxt], idx[q])
            pltpu.make_async_copy(tab_ref.at[idx[p].at[0]], rows[p],
                                  sg[p]).wait()
            pltpu.async_copy(tab_ref.at[idx[q].at[0]], rows[q], sg[q])
            pltpu.sync_copy(rows[p], acc.at[idx[p].at[1]], add=True)

        def loop2(jj, carry):
            halfstep(2 * jj, 0)
            halfstep(2 * jj + 1, 1)
            return carry
        with jax.named_scope("agg_loop"):
            lax.fori_loop(0, nchunk // 2, loop2, 0)
            pltpu.make_async_copy(tab_ref.at[idx0.at[0]], rows0, sg0).wait()
        with jax.named_scope("agg_out"):
            plsc.subcore_barrier()
            pltpu.sync_copy(acc.at[pl.ds(s * rpt, rpt)],
                            out_ref.at[c, pl.ds(s * rpt, rpt)])

    return pl.kernel(
        body,
        out_type=jax.ShapeDtypeStruct((NC, n_acc, H), f32),
        mesh=mesh,
        scratch_types=[
            pltpu.VMEM_SHARED((n_acc, H), f32),
            pltpu.VMEM((2, CH), i32),
            pltpu.VMEM((2, CH), i32),
            pltpu.VMEM((CH, H), f32),
            pltpu.VMEM((CH, H), f32),
            pltpu.SemaphoreType.DMA,
            pltpu.SemaphoreType.DMA,
        ],
    )(table, sd4)


def _relu(x):
    return jnp.maximum(x, 0.0)


def _dot(a, b):
    return jnp.dot(a, b, preferred_element_type=f32)


def _mm_relu_body(x_ref, w_ref, b_ref, o_ref):
    o_ref[...] = _relu(_dot(x_ref[...], w_ref[...]) + b_ref[...])


def _mm_relu(x, w, b, blk):
    n = x.shape[0]
    return pl.pallas_call(
        _mm_relu_body,
        grid=(n // blk,),
        in_specs=[
            pl.BlockSpec((blk, H), lambda i: (i, 0)),
            pl.BlockSpec((H, H), lambda i: (0, 0)),
            pl.BlockSpec((1, H), lambda i: (0, 0)),
        ],
        out_specs=pl.BlockSpec((blk, H), lambda i: (i, 0)),
        out_shape=jax.ShapeDtypeStruct((n, H), f32),
    )(x, w, b.reshape(1, H))


def _layer_body(h_ref, s_ref, inv_ref, ws_ref, wn_ref, b_ref, o_ref):
    agg = (s_ref[0] + s_ref[1]) * inv_ref[...]
    o_ref[...] = _relu(_dot(h_ref[...], ws_ref[...]) + _dot(agg, wn_ref[...])
                       + b_ref[...])


def _fused_layer(h, S, invb, Ws, Wn, b, blk):
    n = h.shape[0]
    return pl.pallas_call(
        _layer_body,
        grid=(n // blk,),
        in_specs=[
            pl.BlockSpec((blk, H), lambda i: (i, 0)),
            pl.BlockSpec((2, blk, H), lambda i: (0, i, 0)),
            pl.BlockSpec((blk, H), lambda i: (i, 0)),
            pl.BlockSpec((H, H), lambda i: (0, 0)),
            pl.BlockSpec((H, H), lambda i: (0, 0)),
            pl.BlockSpec((1, H), lambda i: (0, 0)),
        ],
        out_specs=pl.BlockSpec((blk, H), lambda i: (i, 0)),
        out_shape=jax.ShapeDtypeStruct((n, H), f32),
    )(h, S, invb, Ws, Wn, b.reshape(1, H))


def _dual_body(a_ref, h_ref, w1_ref, w2_ref, b_ref, o_ref):
    o_ref[...] = _relu(_dot(a_ref[...], w1_ref[...]) + _dot(h_ref[...], w2_ref[...])
                       + b_ref[...])


def _dual_mm_relu(a, h, W1, W2, b, blk):
    n = h.shape[0]
    return pl.pallas_call(
        _dual_body,
        grid=(n // blk,),
        in_specs=[
            pl.BlockSpec((blk, H), lambda i: (i, 0)),
            pl.BlockSpec((blk, H), lambda i: (i, 0)),
            pl.BlockSpec((H, H), lambda i: (0, 0)),
            pl.BlockSpec((H, H), lambda i: (0, 0)),
            pl.BlockSpec((1, H), lambda i: (0, 0)),
        ],
        out_specs=pl.BlockSpec((blk, H), lambda i: (i, 0)),
        out_shape=jax.ShapeDtypeStruct((n, H), f32),
    )(a, h, W1, W2, b.reshape(1, H))


def _mlp2_body(x_ref, w_ref, b_ref, o_ref):
    hmid = _relu(_dot(x_ref[...], w_ref[0]) + b_ref[0])
    o_ref[...] = _relu(_dot(hmid, w_ref[1]) + b_ref[1])


def _mlp2(x, W, b, blk):
    n = x.shape[0]
    return pl.pallas_call(
        _mlp2_body,
        grid=(n // blk,),
        in_specs=[
            pl.BlockSpec((blk, H), lambda i: (i, 0)),
            pl.BlockSpec((2, H, H), lambda i: (0, 0, 0)),
            pl.BlockSpec((2, 1, H), lambda i: (0, 0, 0)),
        ],
        out_specs=pl.BlockSpec((blk, H), lambda i: (i, 0)),
        out_shape=jax.ShapeDtypeStruct((n, H), f32),
    )(x, W, b.reshape(2, 1, H))


SEGB = 800


def _up_pool_body(id_ref, d_ref, h_ref, wc_ref, ws_ref, b_ref,
                  o_ref, acc_ref, cnt_ref):
    gj = pl.program_id(0)
    gi = pl.program_id(1)
    ngi = pl.num_programs(1)
    blk = id_ref.shape[0]
    oh = (id_ref[...] == gj * SEGB
          + lax.broadcasted_iota(i32, (blk, SEGB), 1)).astype(f32)
    part = lax.dot_general(oh, d_ref[...], (((0,), (0,)), ((), ())),
                           preferred_element_type=f32)
    pcnt = lax.dot_general(oh, jnp.ones((blk, 1), f32), (((0,), (0,)), ((), ())),
                           preferred_element_type=f32)

    @pl.when(gi == 0)
    def _init():
        acc_ref[...] = jnp.zeros_like(acc_ref)
        cnt_ref[...] = jnp.zeros_like(cnt_ref)

    acc_ref[...] += part
    cnt_ref[...] += pcnt

    @pl.when(gi == ngi - 1)
    def _emit():
        up = acc_ref[...] / jnp.maximum(cnt_ref[...], 1.0)
        o_ref[...] = _relu(_dot(up, wc_ref[...]) + _dot(h_ref[...], ws_ref[...])
                           + b_ref[...])


def _up_pool_mm(ids2d, data, h, Wc, Ws, b, blk):
    n = data.shape[0]
    return pl.pallas_call(
        _up_pool_body,
        grid=(TMAX // SEGB, n // blk),
        in_specs=[
            pl.BlockSpec((blk, 1), lambda gj, gi: (gi, 0)),
            pl.BlockSpec((blk, H), lambda gj, gi: (gi, 0)),
            pl.BlockSpec((SEGB, H), lambda gj, gi: (gj, 0)),
            pl.BlockSpec((H, H), lambda gj, gi: (0, 0)),
            pl.BlockSpec((H, H), lambda gj, gi: (0, 0)),
            pl.BlockSpec((1, H), lambda gj, gi: (0, 0)),
        ],
        out_specs=pl.BlockSpec((SEGB, H), lambda gj, gi: (gj, 0)),
        out_shape=jax.ShapeDtypeStruct((TMAX, H), f32),
        scratch_shapes=[pltpu.VMEM((SEGB, H), f32),
                        pltpu.VMEM((SEGB, 1), f32)],
    )(ids2d, data, h, Wc, Ws, b.reshape(1, H))


def _down_body(id_ref, t_ref, h_ref, wc_ref, ws_ref, b_ref, o_ref, acc_ref):
    gi = pl.program_id(0)
    gj = pl.program_id(1)
    ngj = pl.num_programs(1)
    oh = (id_ref[...] == gj * SEGB
          + lax.broadcasted_iota(i32, (id_ref.shape[0], SEGB), 1)).astype(f32)
    part = _dot(oh, t_ref[...])

    @pl.when(gj == 0)
    def _init():
        acc_ref[...] = jnp.zeros_like(acc_ref)

    acc_ref[...] += part

    @pl.when(gj == ngj - 1)
    def _emit():
        o_ref[...] = _relu(_dot(acc_ref[...], wc_ref[...])
                           + _dot(h_ref[...], ws_ref[...]) + b_ref[...])


def _down_gather_mm(ids2d, table, h, Wc, Ws, b, blk):
    n = h.shape[0]
    return pl.pallas_call(
        _down_body,
        grid=(n // blk, TMAX // SEGB),
        in_specs=[
            pl.BlockSpec((blk, 1), lambda gi, gj: (gi, 0)),
            pl.BlockSpec((SEGB, H), lambda gi, gj: (gj, 0)),
            pl.BlockSpec((blk, H), lambda gi, gj: (gi, 0)),
            pl.BlockSpec((H, H), lambda gi, gj: (0, 0)),
            pl.BlockSpec((H, H), lambda gi, gj: (0, 0)),
            pl.BlockSpec((1, H), lambda gi, gj: (0, 0)),
        ],
        out_specs=pl.BlockSpec((blk, H), lambda gi, gj: (gi, 0)),
        out_shape=jax.ShapeDtypeStruct((n, H), f32),
        scratch_shapes=[pltpu.VMEM((blk, H), f32)],
    )(ids2d, table, h, Wc, Ws, b.reshape(1, H))


def _pool_body(d_ref, id_ref, inv_ref, o_ref):
    i = pl.program_id(0)
    n = pl.num_programs(0)
    oh = (id_ref[...] == lax.broadcasted_iota(i32, (d_ref.shape[0], B), 1)
          ).astype(f32)
    part = lax.dot_general(oh, d_ref[...], (((0,), (0,)), ((), ())),
                           preferred_element_type=f32)

    @pl.when(i == 0)
    def _init():
        o_ref[...] = jnp.zeros_like(o_ref)

    o_ref[...] += part

    @pl.when(i == n - 1)
    def _scale():
        o_ref[...] = o_ref[...] * inv_ref[...]


def _pool_mean(data, ids2d, invb, blk):
    n = data.shape[0]
    return pl.pallas_call(
        _pool_body,
        grid=(n // blk,),
        in_specs=[
            pl.BlockSpec((blk, H), lambda i: (i, 0)),
            pl.BlockSpec((blk, 1), lambda i: (i, 0)),
            pl.BlockSpec((B, H), lambda i: (0, 0)),
        ],
        out_specs=pl.BlockSpec((B, H), lambda i: (0, 0)),
        out_shape=jax.ShapeDtypeStruct((B, H), f32),
    )(data, ids2d, invb)


def _head_body(g0_ref, g1_ref, h2_ref, w1_ref, b1_ref, w2_ref, b2_ref, o_ref):
    hid = _relu(_dot(g0_ref[...], w1_ref[0]) + _dot(g1_ref[...], w1_ref[1])
                + _dot(h2_ref[...], w1_ref[2]) + b1_ref[...])
    o_ref[...] = _dot(hid, w2_ref[...]) + b2_ref[...]


def _head(g0, g1, h2, W1, b1, W2p, b2p):
    return pl.pallas_call(
        _head_body,
        in_specs=[pl.BlockSpec((B, H), lambda: (0, 0))] * 3 + [
            pl.BlockSpec((3, H, H), lambda: (0, 0, 0)),
            pl.BlockSpec((1, H), lambda: (0, 0)),
            pl.BlockSpec((H, H), lambda: (0, 0)),
            pl.BlockSpec((1, H), lambda: (0, 0)),
        ],
        out_specs=pl.BlockSpec((B, H), lambda: (0, 0)),
        out_shape=jax.ShapeDtypeStruct((B, H), f32),
    )(g0, g1, h2, W1.reshape(3, H, H), b1.reshape(1, H), W2p, b2p)


def _gcn_encoder(h, sd4, invdegb, Wself, Wnei, bb):
    for l in range(Wself.shape[0]):
        S = _sc_scatter_sum(h, sd4, NA0)
        h = _fused_layer(h, S, invdegb, Wself[l], Wnei[l], bb[l], 2000)
    return h


def kernel(x, edge_index, batch, assign_index, W_in, b_in, enc_Wself, enc_Wnei,
           enc_b, inter_Wc, inter_Ws, inter_b, head_W1, head_b1, head_W2,
           head_b2):
    src = edge_index[0]
    dst = edge_index[1]

    bids = jnp.arange(B, dtype=i32)
    onehot_b = batch[:, None] == bids[None, :]
    imin = jnp.iinfo(i32).min
    max_func = jnp.max(jnp.where(onehot_b, assign_index[:, None], imin),
                       axis=0)
    nums = max_func + 1
    offsets = jnp.concatenate([jnp.zeros((1,), nums.dtype),
                               jnp.cumsum(nums)[:-1]])
    assign0 = assign_index + offsets[batch]
    bounds = jnp.cumsum(nums)
    assign1 = jnp.sum((jnp.arange(TMAX, dtype=i32)[:, None]
                       >= bounds[None, :]).astype(i32), axis=1)

    srcp = jnp.concatenate([src, jnp.zeros((EPAD - E,), i32)]
                           ).reshape(NW, -1, 1, CH)
    dstp = jnp.concatenate([dst, jnp.full((EPAD - E,), N, i32)]
                           ).reshape(NW, -1, 1, CH)
    sd4 = jnp.concatenate([srcp, dstp], axis=2)

    deg = jax.ops.segment_sum(jnp.ones((E,), f32), dst, num_segments=N)
    invdegb = jnp.broadcast_to((1.0 / jnp.maximum(deg, 1.0))[:, None], (N, H))
    cnt1 = jnp.sum((assign1[:, None] == bids[None, :]).astype(f32), axis=0)
    inv1b = jnp.broadcast_to((1.0 / jnp.maximum(cnt1, 1.0))[:, None], (B, H))
    cntb = jnp.sum(onehot_b.astype(f32), axis=0)
    invbb = jnp.broadcast_to((1.0 / jnp.maximum(cntb, 1.0))[:, None], (B, H))
    batch2d = batch.reshape(N, 1)
    assign1_2d = assign1.reshape(TMAX, 1)
    assign0_2d = assign0.reshape(N, 1)

    x0 = _mm_relu(x, W_in, b_in, 2000)
    x1 = jnp.zeros((TMAX, H), f32)
    x2 = jnp.zeros((B, H), f32)
    h0 = _gcn_encoder(x0, sd4, invdegb, enc_Wself[0], enc_Wnei[0], enc_b[0])
    h1 = _mlp2(x1, enc_Wself[1], enc_b[1], 800)
    h2 = _mlp2(x2, enc_Wself[2], enc_b[2], B)

    for _ in range(2):
        h1n = _up_pool_mm(assign0_2d, h0, h1, inter_Wc[0],
                          inter_Ws[0], inter_b[0], 2000)
        up12 = _pool_mean(h1n, assign1_2d, inv1b, 800)
        h2n = _dual_mm_relu(up12, h2, inter_Wc[1], inter_Ws[1], inter_b[1], B)
        h0 = _down_gather_mm(assign0_2d, h1n, h0, inter_Wc[2], inter_Ws[2],
                             inter_b[2], 2000)
        h1, h2 = h1n, h2n
        h0 = _gcn_encoder(h0, sd4, invdegb, enc_Wself[3], enc_Wnei[3],
                          enc_b[3])
        h1 = _mlp2(h1, enc_Wself[4], enc_b[4], 800)
        h2 = _mlp2(h2, enc_Wself[5], enc_b[5], B)

    g0 = _pool_mean(h0, batch2d, invbb, 2000)
    g1 = _pool_mean(h1, assign1_2d, inv1b, 800)
    W2p = jnp.pad(head_W2, ((0, 0), (0, H - head_W2.shape[1])))
    b2p = jnp.pad(head_b2, (0, H - head_b2.shape[0])).reshape(1, H)
    out = _head(g0, g1, h2, head_W1, head_b1, W2p, b2p)
    return out[:, :head_W2.shape[1]]


# speedup vs baseline: 1.0003x; 1.0003x over previous
import jax
import jax.numpy as jnp
from jax import lax
from jax.experimental import pallas as pl
from jax.experimental.pallas import tpu as pltpu
from jax.experimental.pallas import tpu_sc as plsc

f32 = jnp.float32
i32 = jnp.int32

H = 128
N = 10000
E = 320000
B = 64
TMAX = 3200
NC, NS = 2, 16
NW = NC * NS
CH = 128

EPAD = NW * CH * 80
UPAD = NW * CH * 4
NA0 = 10240
NA1 = 3328


def _sc_scatter_sum(table, sd4, n_acc):
    nchunk = sd4.shape[1]
    assert nchunk % 2 == 0
    rpt = n_acc // NS
    mesh = plsc.VectorSubcoreMesh(core_axis_name="c", subcore_axis_name="s")

    def body(tab_ref, sd_ref, out_ref, acc, idx0, idx1, rows0, rows1,
             sg0, sg1):
        c = lax.axis_index("c")
        s = lax.axis_index("s")
        w = c * NS + s
        with jax.named_scope("agg_zero"):
            pltpu.sync_copy(sd_ref.at[w, 0], idx0)
            pltpu.async_copy(tab_ref.at[idx0.at[0]], rows0, sg0)
            zeros16 = jnp.zeros((16,), f32)

            def zrow(r, carry):
                for j in range(8):
                    rows1[r, pl.ds(j * 16, 16)] = zeros16
                return carry
            lax.fori_loop(0, CH, zrow, 0)
            for k in range(rpt // CH):
                pltpu.sync_copy(rows1, acc.at[pl.ds(s * rpt + k * CH, CH)])
            rem = rpt % CH
            if rem:
                pltpu.sync_copy(rows1.at[pl.ds(0, rem)],
                                acc.at[pl.ds(s * rpt + (rpt // CH) * CH, rem)])
            plsc.subcore_barrier()

        idx = (idx0, idx1)
        rows = (rows0, rows1)
        sg = (sg0, sg1)

        def halfstep(i, p):
            q = 1 - p
            nxt = lax.rem(i + 1, nchunk)
            pltpu.sync_copy(sd_ref.at[w, nxt], idx[q])
            pltpu.make_async_copy(tab_ref.at[idx[p].at[0]], rows[p],
                                  sg[p]).wait()
            pltpu.async_copy(tab_ref.at[idx[q].at[0]], rows[q], sg[q])
            pltpu.sync_copy(rows[p], acc.at[idx[p].at[1]], add=True)

        def loop2(jj, carry):
            halfstep(2 * jj, 0)
            halfstep(2 * jj + 1, 1)
            return carry
        with jax.named_scope("agg_loop"):
            lax.fori_loop(0, nchunk // 2, loop2, 0)
            pltpu.make_async_copy(tab_ref.at[idx0.at[0]], rows0, sg0).wait()
        with jax.named_scope("agg_out"):
            plsc.subcore_barrier()
            pltpu.sync_copy(acc.at[pl.ds(s * rpt, rpt)],
                            out_ref.at[c, pl.ds(s * rpt, rpt)])

    return pl.kernel(
        body,
        out_type=jax.ShapeDtypeStruct((NC, n_acc, H), f32),
        mesh=mesh,
        scratch_types=[
            pltpu.VMEM_SHARED((n_acc, H), f32),
            pltpu.VMEM((2, CH), i32),
            pltpu.VMEM((2, CH), i32),
            pltpu.VMEM((CH, H), f32),
            pltpu.VMEM((CH, H), f32),
            pltpu.SemaphoreType.DMA,
            pltpu.SemaphoreType.DMA,
        ],
    )(table, sd4)


def _relu(x):
    return jnp.maximum(x, 0.0)


def _dot(a, b):
    return jnp.dot(a, b, preferred_element_type=f32)


def _mm_relu_body(x_ref, w_ref, b_ref, o_ref):
    o_ref[...] = _relu(_dot(x_ref[...], w_ref[...]) + b_ref[...])


def _mm_relu(x, w, b, blk):
    n = x.shape[0]
    return pl.pallas_call(
        _mm_relu_body,
        grid=(n // blk,),
        in_specs=[
            pl.BlockSpec((blk, H), lambda i: (i, 0)),
            pl.BlockSpec((H, H), lambda i: (0, 0)),
            pl.BlockSpec((1, H), lambda i: (0, 0)),
        ],
        out_specs=pl.BlockSpec((blk, H), lambda i: (i, 0)),
        out_shape=jax.ShapeDtypeStruct((n, H), f32),
    )(x, w, b.reshape(1, H))


def _layer_body(h_ref, s_ref, inv_ref, ws_ref, wn_ref, b_ref, o_ref):
    agg = (s_ref[0] + s_ref[1]) * inv_ref[...]
    o_ref[...] = _relu(_dot(h_ref[...], ws_ref[...]) + _dot(agg, wn_ref[...])
                       + b_ref[...])


def _fused_layer(h, S, invb, Ws, Wn, b, blk):
    n = h.shape[0]
    return pl.pallas_call(
        _layer_body,
        grid=(n // blk,),
        in_specs=[
            pl.BlockSpec((blk, H), lambda i: (i, 0)),
            pl.BlockSpec((2, blk, H), lambda i: (0, i, 0)),
            pl.BlockSpec((blk, H), lambda i: (i, 0)),
            pl.BlockSpec((H, H), lambda i: (0, 0)),
            pl.BlockSpec((H, H), lambda i: (0, 0)),
            pl.BlockSpec((1, H), lambda i: (0, 0)),
        ],
        out_specs=pl.BlockSpec((blk, H), lambda i: (i, 0)),
        out_shape=jax.ShapeDtypeStruct((n, H), f32),
    )(h, S, invb, Ws, Wn, b.reshape(1, H))


def _dual_body(a_ref, h_ref, w1_ref, w2_ref, b_ref, o_ref):
    o_ref[...] = _relu(_dot(a_ref[...], w1_ref[...]) + _dot(h_ref[...], w2_ref[...])
                       + b_ref[...])


def _dual_mm_relu(a, h, W1, W2, b, blk):
    n = h.shape[0]
    return pl.pallas_call(
        _dual_body,
        grid=(n // blk,),
        in_specs=[
            pl.BlockSpec((blk, H), lambda i: (i, 0)),
            pl.BlockSpec((blk, H), lambda i: (i, 0)),
            pl.BlockSpec((H, H), lambda i: (0, 0)),
            pl.BlockSpec((H, H), lambda i: (0, 0)),
            pl.BlockSpec((1, H), lambda i: (0, 0)),
        ],
        out_specs=pl.BlockSpec((blk, H), lambda i: (i, 0)),
        out_shape=jax.ShapeDtypeStruct((n, H), f32),
    )(a, h, W1, W2, b.reshape(1, H))


def _mlp2_body(x_ref, w_ref, b_ref, o_ref):
    hmid = _relu(_dot(x_ref[...], w_ref[0]) + b_ref[0])
    o_ref[...] = _relu(_dot(hmid, w_ref[1]) + b_ref[1])


def _mlp2(x, W, b, blk):
    n = x.shape[0]
    return pl.pallas_call(
        _mlp2_body,
        grid=(n // blk,),
        in_specs=[
            pl.BlockSpec((blk, H), lambda i: (i, 0)),
            pl.BlockSpec((2, H, H), lambda i: (0, 0, 0)),
            pl.BlockSpec((2, 1, H), lambda i: (0, 0, 0)),
        ],
        out_specs=pl.BlockSpec((blk, H), lambda i: (i, 0)),
        out_shape=jax.ShapeDtypeStruct((n, H), f32),
    )(x, W, b.reshape(2, 1, H))


SEGB = 800


def _up_pool_body(id_ref, d_ref, h_ref, wc_ref, ws_ref, b_ref,
                  o_ref, acc_ref, cnt_ref):
    gj = pl.program_id(0)
    gi = pl.program_id(1)
    ngi = pl.num_programs(1)
    blk = id_ref.shape[0]
    oh = (id_ref[...] == gj * SEGB
          + lax.broadcasted_iota(i32, (blk, SEGB), 1)).astype(f32)
    part = lax.dot_general(oh, d_ref[...], (((0,), (0,)), ((), ())),
                           preferred_element_type=f32)
    pcnt = lax.dot_general(oh, jnp.ones((blk, 1), f32), (((0,), (0,)), ((), ())),
                           preferred_element_type=f32)

    @pl.when(gi == 0)
    def _init():
        acc_ref[...] = jnp.zeros_like(acc_ref)
        cnt_ref[...] = jnp.zeros_like(cnt_ref)

    acc_ref[...] += part
    cnt_ref[...] += pcnt

    @pl.when(gi == ngi - 1)
    def _emit():
        up = acc_ref[...] / jnp.maximum(cnt_ref[...], 1.0)
        o_ref[...] = _relu(_dot(up, wc_ref[...]) + _dot(h_ref[...], ws_ref[...])
                           + b_ref[...])


def _up_pool_mm(ids2d, data, h, Wc, Ws, b, blk):
    n = data.shape[0]
    return pl.pallas_call(
        _up_pool_body,
        grid=(TMAX // SEGB, n // blk),
        in_specs=[
            pl.BlockSpec((blk, 1), lambda gj, gi: (gi, 0)),
            pl.BlockSpec((blk, H), lambda gj, gi: (gi, 0)),
            pl.BlockSpec((SEGB, H), lambda gj, gi: (gj, 0)),
            pl.BlockSpec((H, H), lambda gj, gi: (0, 0)),
            pl.BlockSpec((H, H), lambda gj, gi: (0, 0)),
            pl.BlockSpec((1, H), lambda gj, gi: (0, 0)),
        ],
        out_specs=pl.BlockSpec((SEGB, H), lambda gj, gi: (gj, 0)),
        out_shape=jax.ShapeDtypeStruct((TMAX, H), f32),
        scratch_shapes=[pltpu.VMEM((SEGB, H), f32),
                        pltpu.VMEM((SEGB, 1), f32)],
    )(ids2d, data, h, Wc, Ws, b.reshape(1, H))


def _down_body(id_ref, t_ref, h_ref, wc_ref, ws_ref, b_ref, o_ref, acc_ref):
    gi = pl.program_id(0)
    gj = pl.program_id(1)
    ngj = pl.num_programs(1)
    oh = (id_ref[...] == gj * SEGB
          + lax.broadcasted_iota(i32, (id_ref.shape[0], SEGB), 1)).astype(f32)
    part = _dot(oh, t_ref[...])

    @pl.when(gj == 0)
    def _init():
        acc_ref[...] = jnp.zeros_like(acc_ref)

    acc_ref[...] += part

    @pl.when(gj == ngj - 1)
    def _emit():
        o_ref[...] = _relu(_dot(acc_ref[...], wc_ref[...])
                           + _dot(h_ref[...], ws_ref[...]) + b_ref[...])


def _down_gather_mm(ids2d, table, h, Wc, Ws, b, blk):
    n = h.shape[0]
    return pl.pallas_call(
        _down_body,
        grid=(n // blk, TMAX // SEGB),
        in_specs=[
            pl.BlockSpec((blk, 1), lambda gi, gj: (gi, 0)),
            pl.BlockSpec((SEGB, H), lambda gi, gj: (gj, 0)),
            pl.BlockSpec((blk, H), lambda gi, gj: (gi, 0)),
            pl.BlockSpec((H, H), lambda gi, gj: (0, 0)),
            pl.BlockSpec((H, H), lambda gi, gj: (0, 0)),
            pl.BlockSpec((1, H), lambda gi, gj: (0, 0)),
        ],
        out_specs=pl.BlockSpec((blk, H), lambda gi, gj: (gi, 0)),
        out_shape=jax.ShapeDtypeStruct((n, H), f32),
        scratch_shapes=[pltpu.VMEM((blk, H), f32)],
    )(ids2d, table, h, Wc, Ws, b.reshape(1, H))


def _pool_body(d_ref, id_ref, inv_ref, o_ref):
    i = pl.program_id(0)
    n = pl.num_programs(0)
    oh = (id_ref[...] == lax.broadcasted_iota(i32, (d_ref.shape[0], B), 1)
          ).astype(f32)
    part = lax.dot_general(oh, d_ref[...], (((0,), (0,)), ((), ())),
                           preferred_element_type=f32)

    @pl.when(i == 0)
    def _init():
        o_ref[...] = jnp.zeros_like(o_ref)

    o_ref[...] += part

    @pl.when(i == n - 1)
    def _scale():
        o_ref[...] = o_ref[...] * inv_ref[...]


def _pool_mean(data, ids2d, invb, blk):
    n = data.shape[0]
    return pl.pallas_call(
        _pool_body,
        grid=(n // blk,),
        in_specs=[
            pl.BlockSpec((blk, H), lambda i: (i, 0)),
            pl.BlockSpec((blk, 1), lambda i: (i, 0)),
            pl.BlockSpec((B, H), lambda i: (0, 0)),
        ],
        out_specs=pl.BlockSpec((B, H), lambda i: (0, 0)),
        out_shape=jax.ShapeDtypeStruct((B, H), f32),
    )(data, ids2d, invb)


def _head_body(g0_ref, g1_ref, h2_ref, w1_ref, b1_ref, w2_ref, b2_ref, o_ref):
    hid = _relu(_dot(g0_ref[...], w1_ref[0]) + _dot(g1_ref[...], w1_ref[1])
                + _dot(h2_ref[...], w1_ref[2]) + b1_ref[...])
    o_ref[...] = _dot(hid, w2_ref[...]) + b2_ref[...]


def _head(g0, g1, h2, W1, b1, W2p, b2p):
    return pl.pallas_call(
        _head_body,
        in_specs=[pl.BlockSpec((B, H), lambda: (0, 0))] * 3 + [
            pl.BlockSpec((3, H, H), lambda: (0, 0, 0)),
            pl.BlockSpec((1, H), lambda: (0, 0)),
            pl.BlockSpec((H, H), lambda: (0, 0)),
            pl.BlockSpec((1, H), lambda: (0, 0)),
        ],
        out_specs=pl.BlockSpec((B, H), lambda: (0, 0)),
        out_shape=jax.ShapeDtypeStruct((B, H), f32),
    )(g0, g1, h2, W1.reshape(3, H, H), b1.reshape(1, H), W2p, b2p)


def _gcn_encoder(h, sd4, invdegb, Wself, Wnei, bb):
    for l in range(Wself.shape[0]):
        S = _sc_scatter_sum(h, sd4, NA0)
        h = _fused_layer(h, S, invdegb, Wself[l], Wnei[l], bb[l], 2000)
    return h


def kernel(x, edge_index, batch, assign_index, W_in, b_in, enc_Wself, enc_Wnei,
           enc_b, inter_Wc, inter_Ws, inter_b, head_W1, head_b1, head_W2,
           head_b2):
    src = edge_index[0]
    dst = edge_index[1]

    bids = jnp.arange(B, dtype=i32)
    onehot_b = batch[:, None] == bids[None, :]
    imin = jnp.iinfo(i32).min
    max_func = jnp.max(jnp.where(onehot_b, assign_index[:, None], imin),
                       axis=0)
    nums = max_func + 1
    offsets = jnp.concatenate([jnp.zeros((1,), nums.dtype),
                               jnp.cumsum(nums)[:-1]])
    assign0 = assign_index + offsets[batch]
    bounds = jnp.cumsum(nums)
    assign1 = jnp.sum((jnp.arange(TMAX, dtype=i32)[:, None]
                       >= bounds[None, :]).astype(i32), axis=1)

    srcp = jnp.concatenate([src, jnp.zeros((EPAD - E,), i32)]
                           ).reshape(NW, -1, 1, CH)
    pad_dst = N + jnp.arange(EPAD - E, dtype=i32) % (NA0 - N)
    dstp = jnp.concatenate([dst, pad_dst]).reshape(NW, -1, 1, CH)
    sd4 = jnp.concatenate([srcp, dstp], axis=2)

    deg = jax.ops.segment_sum(jnp.ones((E,), f32), dst, num_segments=N)
    invdegb = jnp.broadcast_to((1.0 / jnp.maximum(deg, 1.0))[:, None], (N, H))
    cnt1 = jnp.sum((assign1[:, None] == bids[None, :]).astype(f32), axis=0)
    inv1b = jnp.broadcast_to((1.0 / jnp.maximum(cnt1, 1.0))[:, None], (B, H))
    cntb = jnp.sum(onehot_b.astype(f32), axis=0)
    invbb = jnp.broadcast_to((1.0 / jnp.maximum(cntb, 1.0))[:, None], (B, H))
    batch2d = batch.reshape(N, 1)
    assign1_2d = assign1.reshape(TMAX, 1)
    assign0_2d = assign0.reshape(N, 1)

    x0 = _mm_relu(x, W_in, b_in, 2000)
    x1 = jnp.zeros((TMAX, H), f32)
    x2 = jnp.zeros((B, H), f32)
    h0 = _gcn_encoder(x0, sd4, invdegb, enc_Wself[0], enc_Wnei[0], enc_b[0])
    h1 = _mlp2(x1, enc_Wself[1], enc_b[1], 800)
    h2 = _mlp2(x2, enc_Wself[2], enc_b[2], B)

    for _ in range(2):
        h1n = _up_pool_mm(assign0_2d, h0, h1, inter_Wc[0],
                          inter_Ws[0], inter_b[0], 2000)
        up12 = _pool_mean(h1n, assign1_2d, inv1b, 800)
        h2n = _dual_mm_relu(up12, h2, inter_Wc[1], inter_Ws[1], inter_b[1], B)
        h0 = _down_gather_mm(assign0_2d, h1n, h0, inter_Wc[2], inter_Ws[2],
                             inter_b[2], 2000)
        h1, h2 = h1n, h2n
        h0 = _gcn_encoder(h0, sd4, invdegb, enc_Wself[3], enc_Wnei[3],
                          enc_b[3])
        h1 = _mlp2(h1, enc_Wself[4], enc_b[4], 800)
        h2 = _mlp2(h2, enc_Wself[5], enc_b[5], B)

    g0 = _pool_mean(h0, batch2d, invbb, 2000)
    g1 = _pool_mean(h1, assign1_2d, inv1b, 800)
    W2p = jnp.pad(head_W2, ((0, 0), (0, H - head_W2.shape[1])))
    b2p = jnp.pad(head_b2, (0, H - head_b2.shape[0])).reshape(1, H)
    out = _head(g0, g1, h2, head_W1, head_b1, W2p, b2p)
    return out[:, :head_W2.shape[1]]


# speedup vs baseline: 1.0883x; 1.0879x over previous
import jax
import jax.numpy as jnp
from jax import lax
from jax.experimental import pallas as pl
from jax.experimental.pallas import tpu as pltpu
from jax.experimental.pallas import tpu_sc as plsc

f32 = jnp.float32
i32 = jnp.int32

H = 128
N = 10000
E = 320000
B = 64
TMAX = 3200
NC, NS = 2, 16
NW = NC * NS
CH = 128

EPAD = NW * CH * 80
UPAD = NW * CH * 4
NA0 = 10240
NA1 = 3328


def _sc_scatter_sum(table, sd4, n_acc):
    nchunk = sd4.shape[1]
    assert nchunk % 2 == 0
    rpt = n_acc // NS
    mesh = plsc.VectorSubcoreMesh(core_axis_name="c", subcore_axis_name="s")

    def body(tab_ref, sd_ref, out_ref, acc, idx0, idx1, rows0, rows1,
             sg0, sg1):
        c = lax.axis_index("c")
        s = lax.axis_index("s")
        w = c * NS + s
        with jax.named_scope("agg_zero"):
            pltpu.sync_copy(sd_ref.at[w, 0], idx0)
            pltpu.async_copy(tab_ref.at[idx0.at[0]], rows0, sg0)
            zeros16 = jnp.zeros((16,), f32)

            def zrow(r, carry):
                for j in range(8):
                    rows1[r, pl.ds(j * 16, 16)] = zeros16
                return carry
            lax.fori_loop(0, CH, zrow, 0)
            for k in range(rpt // CH):
                pltpu.sync_copy(rows1, acc.at[pl.ds(s * rpt + k * CH, CH)])
            rem = rpt % CH
            if rem:
                pltpu.sync_copy(rows1.at[pl.ds(0, rem)],
                                acc.at[pl.ds(s * rpt + (rpt // CH) * CH, rem)])
            plsc.subcore_barrier()

        idx = (idx0, idx1)
        rows = (rows0, rows1)
        sg = (sg0, sg1)

        def halfstep(i, p):
            q = 1 - p
            nxt = lax.rem(i + 1, nchunk)
            pltpu.sync_copy(sd_ref.at[w, nxt], idx[q])
            pltpu.make_async_copy(tab_ref.at[idx[p].at[0]], rows[p],
                                  sg[p]).wait()
            pltpu.async_copy(tab_ref.at[idx[q].at[0]], rows[q], sg[q])
            pltpu.sync_copy(rows[p], acc.at[idx[p].at[1]], add=True)

        def loop2(jj, carry):
            halfstep(2 * jj, 0)
            halfstep(2 * jj + 1, 1)
            return carry
        with jax.named_scope("agg_loop"):
            lax.fori_loop(0, nchunk // 2, loop2, 0)
            pltpu.make_async_copy(tab_ref.at[idx0.at[0]], rows0, sg0).wait()
        with jax.named_scope("agg_out"):
            plsc.subcore_barrier()
            pltpu.sync_copy(acc.at[pl.ds(s * rpt, rpt)],
                            out_ref.at[c, pl.ds(s * rpt, rpt)])

    return pl.kernel(
        body,
        out_type=jax.ShapeDtypeStruct((NC, n_acc, H), f32),
        mesh=mesh,
        scratch_types=[
            pltpu.VMEM_SHARED((n_acc, H), f32),
            pltpu.VMEM((2, CH), i32),
            pltpu.VMEM((2, CH), i32),
            pltpu.VMEM((CH, H), f32),
            pltpu.VMEM((CH, H), f32),
            pltpu.SemaphoreType.DMA,
            pltpu.SemaphoreType.DMA,
        ],
    )(table, sd4)


def _relu(x):
    return jnp.maximum(x, 0.0)


def _dot(a, b):
    return jnp.dot(a, b, preferred_element_type=f32)


def _mm_relu_body(x_ref, w_ref, b_ref, o_ref):
    o_ref[...] = _relu(_dot(x_ref[...], w_ref[...]) + b_ref[...])


def _mm_relu(x, w, b, blk):
    n = x.shape[0]
    return pl.pallas_call(
        _mm_relu_body,
        grid=(n // blk,),
        in_specs=[
            pl.BlockSpec((blk, H), lambda i: (i, 0)),
            pl.BlockSpec((H, H), lambda i: (0, 0)),
            pl.BlockSpec((1, H), lambda i: (0, 0)),
        ],
        out_specs=pl.BlockSpec((blk, H), lambda i: (i, 0)),
        out_shape=jax.ShapeDtypeStruct((n, H), f32),
    )(x, w, b.reshape(1, H))


def _layer_body(h_ref, s_ref, inv_ref, ws_ref, wn_ref, b_ref, o_ref):
    agg = (s_ref[0] + s_ref[1]) * inv_ref[...]
    o_ref[...] = _relu(_dot(h_ref[...], ws_ref[...]) + _dot(agg, wn_ref[...])
                       + b_ref[...])


def _fused_layer(h, S, invb, Ws, Wn, b, blk):
    n = h.shape[0]
    return pl.pallas_call(
        _layer_body,
        grid=(n // blk,),
        in_specs=[
            pl.BlockSpec((blk, H), lambda i: (i, 0)),
            pl.BlockSpec((2, blk, H), lambda i: (0, i, 0)),
            pl.BlockSpec((blk, H), lambda i: (i, 0)),
            pl.BlockSpec((H, H), lambda i: (0, 0)),
            pl.BlockSpec((H, H), lambda i: (0, 0)),
            pl.BlockSpec((1, H), lambda i: (0, 0)),
        ],
        out_specs=pl.BlockSpec((blk, H), lambda i: (i, 0)),
        out_shape=jax.ShapeDtypeStruct((n, H), f32),
    )(h, S, invb, Ws, Wn, b.reshape(1, H))


def _dual_body(a_ref, h_ref, w1_ref, w2_ref, b_ref, o_ref):
    o_ref[...] = _relu(_dot(a_ref[...], w1_ref[...]) + _dot(h_ref[...], w2_ref[...])
                       + b_ref[...])


def _dual_mm_relu(a, h, W1, W2, b, blk):
    n = h.shape[0]
    return pl.pallas_call(
        _dual_body,
        grid=(n // blk,),
        in_specs=[
            pl.BlockSpec((blk, H), lambda i: (i, 0)),
            pl.BlockSpec((blk, H), lambda i: (i, 0)),
            pl.BlockSpec((H, H), lambda i: (0, 0)),
            pl.BlockSpec((H, H), lambda i: (0, 0)),
            pl.BlockSpec((1, H), lambda i: (0, 0)),
        ],
        out_specs=pl.BlockSpec((blk, H), lambda i: (i, 0)),
        out_shape=jax.ShapeDtypeStruct((n, H), f32),
    )(a, h, W1, W2, b.reshape(1, H))


def _mlp2_body(x_ref, w_ref, b_ref, o_ref):
    hmid = _relu(_dot(x_ref[...], w_ref[0]) + b_ref[0])
    o_ref[...] = _relu(_dot(hmid, w_ref[1]) + b_ref[1])


def _mlp2(x, W, b, blk):
    n = x.shape[0]
    return pl.pallas_call(
        _mlp2_body,
        grid=(n // blk,),
        in_specs=[
            pl.BlockSpec((blk, H), lambda i: (i, 0)),
            pl.BlockSpec((2, H, H), lambda i: (0, 0, 0)),
            pl.BlockSpec((2, 1, H), lambda i: (0, 0, 0)),
        ],
        out_specs=pl.BlockSpec((blk, H), lambda i: (i, 0)),
        out_shape=jax.ShapeDtypeStruct((n, H), f32),
    )(x, W, b.reshape(2, 1, H))


SEGB = 800


def _up_pool_body(id_ref, d_ref, h_ref, wc_ref, ws_ref, b_ref,
                  o_ref, acc_ref, cnt_ref):
    gj = pl.program_id(0)
    gi = pl.program_id(1)
    ngi = pl.num_programs(1)
    blk = id_ref.shape[0]
    oh = (id_ref[...] == gj * SEGB
          + lax.broadcasted_iota(i32, (blk, SEGB), 1)).astype(f32)
    part = lax.dot_general(oh, d_ref[...], (((0,), (0,)), ((), ())),
                           preferred_element_type=f32)
    pcnt = lax.dot_general(oh, jnp.ones((blk, 1), f32), (((0,), (0,)), ((), ())),
                           preferred_element_type=f32)

    @pl.when(gi == 0)
    def _init():
        acc_ref[...] = jnp.zeros_like(acc_ref)
        cnt_ref[...] = jnp.zeros_like(cnt_ref)

    acc_ref[...] += part
    cnt_ref[...] += pcnt

    @pl.when(gi == ngi - 1)
    def _emit():
        up = acc_ref[...] / jnp.maximum(cnt_ref[...], 1.0)
        o_ref[...] = _relu(_dot(up, wc_ref[...]) + _dot(h_ref[...], ws_ref[...])
                           + b_ref[...])


def _up_pool_mm(ids2d, data, h, Wc, Ws, b, blk):
    n = data.shape[0]
    return pl.pallas_call(
        _up_pool_body,
        grid=(TMAX // SEGB, n // blk),
        in_specs=[
            pl.BlockSpec((blk, 1), lambda gj, gi: (gi, 0)),
            pl.BlockSpec((blk, H), lambda gj, gi: (gi, 0)),
            pl.BlockSpec((SEGB, H), lambda gj, gi: (gj, 0)),
            pl.BlockSpec((H, H), lambda gj, gi: (0, 0)),
            pl.BlockSpec((H, H), lambda gj, gi: (0, 0)),
            pl.BlockSpec((1, H), lambda gj, gi: (0, 0)),
        ],
        out_specs=pl.BlockSpec((SEGB, H), lambda gj, gi: (gj, 0)),
        out_shape=jax.ShapeDtypeStruct((TMAX, H), f32),
        scratch_shapes=[pltpu.VMEM((SEGB, H), f32),
                        pltpu.VMEM((SEGB, 1), f32)],
    )(ids2d, data, h, Wc, Ws, b.reshape(1, H))


def _down_body(id_ref, t_ref, h_ref, wc_ref, ws_ref, b_ref, o_ref, acc_ref):
    gi = pl.program_id(0)
    gj = pl.program_id(1)
    ngj = pl.num_programs(1)
    oh = (id_ref[...] == gj * SEGB
          + lax.broadcasted_iota(i32, (id_ref.shape[0], SEGB), 1)).astype(f32)
    part = _dot(oh, t_ref[...])

    @pl.when(gj == 0)
    def _init():
        acc_ref[...] = jnp.zeros_like(acc_ref)

    acc_ref[...] += part

    @pl.when(gj == ngj - 1)
    def _emit():
        o_ref[...] = _relu(_dot(acc_ref[...], wc_ref[...])
                           + _dot(h_ref[...], ws_ref[...]) + b_ref[...])


def _down_gather_mm(ids2d, table, h, Wc, Ws, b, blk):
    n = h.shape[0]
    return pl.pallas_call(
        _down_body,
        grid=(n // blk, TMAX // SEGB),
        in_specs=[
            pl.BlockSpec((blk, 1), lambda gi, gj: (gi, 0)),
            pl.BlockSpec((SEGB, H), lambda gi, gj: (gj, 0)),
            pl.BlockSpec((blk, H), lambda gi, gj: (gi, 0)),
            pl.BlockSpec((H, H), lambda gi, gj: (0, 0)),
            pl.BlockSpec((H, H), lambda gi, gj: (0, 0)),
            pl.BlockSpec((1, H), lambda gi, gj: (0, 0)),
        ],
        out_specs=pl.BlockSpec((blk, H), lambda gi, gj: (gi, 0)),
        out_shape=jax.ShapeDtypeStruct((n, H), f32),
        scratch_shapes=[pltpu.VMEM((blk, H), f32)],
    )(ids2d, table, h, Wc, Ws, b.reshape(1, H))


def _pool_body(d_ref, id_ref, inv_ref, o_ref):
    i = pl.program_id(0)
    n = pl.num_programs(0)
    oh = (id_ref[...] == lax.broadcasted_iota(i32, (d_ref.shape[0], B), 1)
          ).astype(f32)
    part = lax.dot_general(oh, d_ref[...], (((0,), (0,)), ((), ())),
                           preferred_element_type=f32)

    @pl.when(i == 0)
    def _init():
        o_ref[...] = jnp.zeros_like(o_ref)

    o_ref[...] += part

    @pl.when(i == n - 1)
    def _scale():
        o_ref[...] = o_ref[...] * inv_ref[...]


def _pool_mean(data, ids2d, invb, blk):
    n = data.shape[0]
    return pl.pallas_call(
        _pool_body,
        grid=(n // blk,),
        in_specs=[
            pl.BlockSpec((blk, H), lambda i: (i, 0)),
            pl.BlockSpec((blk, 1), lambda i: (i, 0)),
            pl.BlockSpec((B, H), lambda i: (0, 0)),
        ],
        out_specs=pl.BlockSpec((B, H), lambda i: (0, 0)),
        out_shape=jax.ShapeDtypeStruct((B, H), f32),
    )(data, ids2d, invb)


def _head_body(g0_ref, g1_ref, h2_ref, w1_ref, b1_ref, w2_ref, b2_ref, o_ref):
    hid = _relu(_dot(g0_ref[...], w1_ref[0]) + _dot(g1_ref[...], w1_ref[1])
                + _dot(h2_ref[...], w1_ref[2]) + b1_ref[...])
    o_ref[...] = _dot(hid, w2_ref[...]) + b2_ref[...]


def _head(g0, g1, h2, W1, b1, W2p, b2p):
    return pl.pallas_call(
        _head_body,
        in_specs=[pl.BlockSpec((B, H), lambda: (0, 0))] * 3 + [
            pl.BlockSpec((3, H, H), lambda: (0, 0, 0)),
            pl.BlockSpec((1, H), lambda: (0, 0)),
            pl.BlockSpec((H, H), lambda: (0, 0)),
            pl.BlockSpec((1, H), lambda: (0, 0)),
        ],
        out_specs=pl.BlockSpec((B, H), lambda: (0, 0)),
        out_shape=jax.ShapeDtypeStruct((B, H), f32),
    )(g0, g1, h2, W1.reshape(3, H, H), b1.reshape(1, H), W2p, b2p)


def _gcn_encoder(h, sd4, invdegb, Wself, Wnei, bb):
    for l in range(Wself.shape[0]):
        S = _sc_scatter_sum(h, sd4, NA0)
        h = _fused_layer(h, S, invdegb, Wself[l], Wnei[l], bb[l], 2000)
    return h


def kernel(x, edge_index, batch, assign_index, W_in, b_in, enc_Wself, enc_Wnei,
           enc_b, inter_Wc, inter_Ws, inter_b, head_W1, head_b1, head_W2,
           head_b2):
    src = edge_index[0]
    dst = edge_index[1]

    bids = jnp.arange(B, dtype=i32)
    onehot_b = batch[:, None] == bids[None, :]
    imin = jnp.iinfo(i32).min
    max_func = jnp.max(jnp.where(onehot_b, assign_index[:, None], imin),
                       axis=0)
    nums = max_func + 1
    offsets = jnp.concatenate([jnp.zeros((1,), nums.dtype),
                               jnp.cumsum(nums)[:-1]])
    assign0 = assign_index + offsets[batch]
    bounds = jnp.cumsum(nums)
    assign1 = jnp.sum((jnp.arange(TMAX, dtype=i32)[:, None]
                       >= bounds[None, :]).astype(i32), axis=1)

    srcp = jnp.concatenate([src, jnp.zeros((EPAD - E,), i32)]
                           ).reshape(-1, NW, 1, CH).swapaxes(0, 1)
    pad_dst = N + jnp.arange(EPAD - E, dtype=i32) % (NA0 - N)
    dstp = jnp.concatenate([dst, pad_dst]
                           ).reshape(-1, NW, 1, CH).swapaxes(0, 1)
    sd4 = jnp.concatenate([srcp, dstp], axis=2)

    deg = jax.ops.segment_sum(jnp.ones((E,), f32), dst, num_segments=N)
    invdegb = jnp.broadcast_to((1.0 / jnp.maximum(deg, 1.0))[:, None], (N, H))
    cnt1 = jnp.sum((assign1[:, None] == bids[None, :]).astype(f32), axis=0)
    inv1b = jnp.broadcast_to((1.0 / jnp.maximum(cnt1, 1.0))[:, None], (B, H))
    cntb = jnp.sum(onehot_b.astype(f32), axis=0)
    invbb = jnp.broadcast_to((1.0 / jnp.maximum(cntb, 1.0))[:, None], (B, H))
    batch2d = batch.reshape(N, 1)
    assign1_2d = assign1.reshape(TMAX, 1)
    assign0_2d = assign0.reshape(N, 1)

    x0 = _mm_relu(x, W_in, b_in, 2000)
    x1 = jnp.zeros((TMAX, H), f32)
    x2 = jnp.zeros((B, H), f32)
    h0 = _gcn_encoder(x0, sd4, invdegb, enc_Wself[0], enc_Wnei[0], enc_b[0])
    h1 = _mlp2(x1, enc_Wself[1], enc_b[1], 800)
    h2 = _mlp2(x2, enc_Wself[2], enc_b[2], B)

    for _ in range(2):
        h1n = _up_pool_mm(assign0_2d, h0, h1, inter_Wc[0],
                          inter_Ws[0], inter_b[0], 2000)
        up12 = _pool_mean(h1n, assign1_2d, inv1b, 800)
        h2n = _dual_mm_relu(up12, h2, inter_Wc[1], inter_Ws[1], inter_b[1], B)
        h0 = _down_gather_mm(assign0_2d, h1n, h0, inter_Wc[2], inter_Ws[2],
                             inter_b[2], 2000)
        h1, h2 = h1n, h2n
        h0 = _gcn_encoder(h0, sd4, invdegb, enc_Wself[3], enc_Wnei[3],
                          enc_b[3])
        h1 = _mlp2(h1, enc_Wself[4], enc_b[4], 800)
        h2 = _mlp2(h2, enc_Wself[5], enc_b[5], B)

    g0 = _pool_mean(h0, batch2d, invbb, 2000)
    g1 = _pool_mean(h1, assign1_2d, inv1b, 800)
    W2p = jnp.pad(head_W2, ((0, 0), (0, H - head_W2.shape[1])))
    b2p = jnp.pad(head_b2, (0, H - head_b2.shape[0])).reshape(1, H)
    out = _head(g0, g1, h2, head_W1, head_b1, W2p, b2p)
    return out[:, :head_W2.shape[1]]


# speedup vs baseline: 2.2940x; 2.1080x over previous
import jax
import jax.numpy as jnp
from jax import lax
from jax.experimental import pallas as pl
from jax.experimental.pallas import tpu as pltpu
from jax.experimental.pallas import tpu_sc as plsc

f32 = jnp.float32
i32 = jnp.int32

H = 128
N = 10000
E = 320000
B = 64
TMAX = 3200
NC, NS = 2, 16
NW = NC * NS
CH = 100
NA0 = 10240


def _sc_scatter_sum(table, sd4, n_acc):
    nchunk = sd4.shape[1]
    assert nchunk % 2 == 0
    rpt = n_acc // NS
    mesh = plsc.VectorSubcoreMesh(core_axis_name="c", subcore_axis_name="s")

    def body(tab_ref, sd_ref, out_ref, acc, idx0, idx1, rows0, rows1,
             sg0, sg1):
        c = lax.axis_index("c")
        s = lax.axis_index("s")
        w = c * NS + s
        with jax.named_scope("agg_zero"):
            pltpu.sync_copy(sd_ref.at[w, 0], idx0)
            pltpu.async_copy(tab_ref.at[idx0.at[0]], rows0, sg0)
            zeros16 = jnp.zeros((16,), f32)

            def zrow(r, carry):
                for j in range(8):
                    rows1[r, pl.ds(j * 16, 16)] = zeros16
                return carry
            lax.fori_loop(0, CH, zrow, 0)
            for k in range(rpt // CH):
                pltpu.sync_copy(rows1, acc.at[pl.ds(s * rpt + k * CH, CH)])
            rem = rpt % CH
            if rem:
                pltpu.sync_copy(rows1.at[pl.ds(0, rem)],
                                acc.at[pl.ds(s * rpt + (rpt // CH) * CH, rem)])
            plsc.subcore_barrier()

        idx = (idx0, idx1)
        rows = (rows0, rows1)
        sg = (sg0, sg1)

        def halfstep(i, p):
            q = 1 - p
            nxt = lax.rem(i + 1, nchunk)
            pltpu.sync_copy(sd_ref.at[w, nxt], idx[q])
            pltpu.make_async_copy(tab_ref.at[idx[p].at[0]], rows[p],
                                  sg[p]).wait()
            pltpu.async_copy(tab_ref.at[idx[q].at[0]], rows[q], sg[q])
            pltpu.sync_copy(rows[p], acc.at[idx[p].at[1]], add=True)

        def loop2(jj, carry):
            halfstep(2 * jj, 0)
            halfstep(2 * jj + 1, 1)
            return carry
        with jax.named_scope("agg_loop"):
            lax.fori_loop(0, nchunk // 2, loop2, 0)
            pltpu.make_async_copy(tab_ref.at[idx0.at[0]], rows0, sg0).wait()
        with jax.named_scope("agg_out"):
            plsc.subcore_barrier()
            pltpu.sync_copy(acc.at[pl.ds(s * rpt, rpt)],
                            out_ref.at[c, pl.ds(s * rpt, rpt)])

    return pl.kernel(
        body,
        out_type=jax.ShapeDtypeStruct((NC, n_acc, H), f32),
        mesh=mesh,
        scratch_types=[
            pltpu.VMEM_SHARED((n_acc, H), f32),
            pltpu.VMEM((2, CH), i32),
            pltpu.VMEM((2, CH), i32),
            pltpu.VMEM((CH, H), f32),
            pltpu.VMEM((CH, H), f32),
            pltpu.SemaphoreType.DMA,
            pltpu.SemaphoreType.DMA,
        ],
    )(table, sd4)


def _relu(x):
    return jnp.maximum(x, 0.0)


def _dot(a, b):
    return jnp.dot(a, b, preferred_element_type=f32)


def _mm_relu_body(x_ref, w_ref, b_ref, o_ref):
    o_ref[...] = _relu(_dot(x_ref[...], w_ref[...]) + b_ref[...])


def _mm_relu(x, w, b, blk):
    n = x.shape[0]
    return pl.pallas_call(
        _mm_relu_body,
        grid=(n // blk,),
        in_specs=[
            pl.BlockSpec((blk, H), lambda i: (i, 0)),
            pl.BlockSpec((H, H), lambda i: (0, 0)),
            pl.BlockSpec((1, H), lambda i: (0, 0)),
        ],
        out_specs=pl.BlockSpec((blk, H), lambda i: (i, 0)),
        out_shape=jax.ShapeDtypeStruct((n, H), f32),
    )(x, w, b.reshape(1, H))


def _layer_body(h_ref, s_ref, inv_ref, ws_ref, wn_ref, b_ref, o_ref):
    agg = (s_ref[0] + s_ref[1]) * inv_ref[...]
    o_ref[...] = _relu(_dot(h_ref[...], ws_ref[...]) + _dot(agg, wn_ref[...])
                       + b_ref[...])


def _fused_layer(h, S, invb, Ws, Wn, b, blk):
    n = h.shape[0]
    return pl.pallas_call(
        _layer_body,
        grid=(n // blk,),
        in_specs=[
            pl.BlockSpec((blk, H), lambda i: (i, 0)),
            pl.BlockSpec((2, blk, H), lambda i: (0, i, 0)),
            pl.BlockSpec((blk, H), lambda i: (i, 0)),
            pl.BlockSpec((H, H), lambda i: (0, 0)),
            pl.BlockSpec((H, H), lambda i: (0, 0)),
            pl.BlockSpec((1, H), lambda i: (0, 0)),
        ],
        out_specs=pl.BlockSpec((blk, H), lambda i: (i, 0)),
        out_shape=jax.ShapeDtypeStruct((n, H), f32),
    )(h, S, invb, Ws, Wn, b.reshape(1, H))


def _dual_body(a_ref, h_ref, w1_ref, w2_ref, b_ref, o_ref):
    o_ref[...] = _relu(_dot(a_ref[...], w1_ref[...]) + _dot(h_ref[...], w2_ref[...])
                       + b_ref[...])


def _dual_mm_relu(a, h, W1, W2, b, blk):
    n = h.shape[0]
    return pl.pallas_call(
        _dual_body,
        grid=(n // blk,),
        in_specs=[
            pl.BlockSpec((blk, H), lambda i: (i, 0)),
            pl.BlockSpec((blk, H), lambda i: (i, 0)),
            pl.BlockSpec((H, H), lambda i: (0, 0)),
            pl.BlockSpec((H, H), lambda i: (0, 0)),
            pl.BlockSpec((1, H), lambda i: (0, 0)),
        ],
        out_specs=pl.BlockSpec((blk, H), lambda i: (i, 0)),
        out_shape=jax.ShapeDtypeStruct((n, H), f32),
    )(a, h, W1, W2, b.reshape(1, H))


def _mlp2_body(x_ref, w_ref, b_ref, o_ref):
    hmid = _relu(_dot(x_ref[...], w_ref[0]) + b_ref[0])
    o_ref[...] = _relu(_dot(hmid, w_ref[1]) + b_ref[1])


def _mlp2(x, W, b, blk):
    n = x.shape[0]
    return pl.pallas_call(
        _mlp2_body,
        grid=(n // blk,),
        in_specs=[
            pl.BlockSpec((blk, H), lambda i: (i, 0)),
            pl.BlockSpec((2, H, H), lambda i: (0, 0, 0)),
            pl.BlockSpec((2, 1, H), lambda i: (0, 0, 0)),
        ],
        out_specs=pl.BlockSpec((blk, H), lambda i: (i, 0)),
        out_shape=jax.ShapeDtypeStruct((n, H), f32),
    )(x, W, b.reshape(2, 1, H))


SEGB = 800


def _up_pool_body(id_ref, d_ref, h_ref, wc_ref, ws_ref, b_ref,
                  o_ref, acc_ref, cnt_ref):
    gj = pl.program_id(0)
    gi = pl.program_id(1)
    ngi = pl.num_programs(1)
    blk = id_ref.shape[0]
    oh = (id_ref[...] == gj * SEGB
          + lax.broadcasted_iota(i32, (blk, SEGB), 1)).astype(f32)
    part = lax.dot_general(oh, d_ref[...], (((0,), (0,)), ((), ())),
                           preferred_element_type=f32)
    pcnt = lax.dot_general(oh, jnp.ones((blk, 1), f32), (((0,), (0,)), ((), ())),
                           preferred_element_type=f32)

    @pl.when(gi == 0)
    def _init():
        acc_ref[...] = jnp.zeros_like(acc_ref)
        cnt_ref[...] = jnp.zeros_like(cnt_ref)

    acc_ref[...] += part
    cnt_ref[...] += pcnt

    @pl.when(gi == ngi - 1)
    def _emit():
        up = acc_ref[...] / jnp.maximum(cnt_ref[...], 1.0)
        o_ref[...] = _relu(_dot(up, wc_ref[...]) + _dot(h_ref[...], ws_ref[...])
                           + b_ref[...])


def _up_pool_mm(ids2d, data, h, Wc, Ws, b, blk):
    n = data.shape[0]
    return pl.pallas_call(
        _up_pool_body,
        grid=(TMAX // SEGB, n // blk),
        in_specs=[
            pl.BlockSpec((blk, 1), lambda gj, gi: (gi, 0)),
            pl.BlockSpec((blk, H), lambda gj, gi: (gi, 0)),
            pl.BlockSpec((SEGB, H), lambda gj, gi: (gj, 0)),
            pl.BlockSpec((H, H), lambda gj, gi: (0, 0)),
            pl.BlockSpec((H, H), lambda gj, gi: (0, 0)),
            pl.BlockSpec((1, H), lambda gj, gi: (0, 0)),
        ],
        out_specs=pl.BlockSpec((SEGB, H), lambda gj, gi: (gj, 0)),
        out_shape=jax.ShapeDtypeStruct((TMAX, H), f32),
        scratch_shapes=[pltpu.VMEM((SEGB, H), f32),
                        pltpu.VMEM((SEGB, 1), f32)],
    )(ids2d, data, h, Wc, Ws, b.reshape(1, H))


def _down_body(id_ref, t_ref, h_ref, wc_ref, ws_ref, b_ref, o_ref, acc_ref):
    gi = pl.program_id(0)
    gj = pl.program_id(1)
    ngj = pl.num_programs(1)
    oh = (id_ref[...] == gj * SEGB
          + lax.broadcasted_iota(i32, (id_ref.shape[0], SEGB), 1)).astype(f32)
    part = _dot(oh, t_ref[...])

    @pl.when(gj == 0)
    def _init():
        acc_ref[...] = jnp.zeros_like(acc_ref)

    acc_ref[...] += part

    @pl.when(gj == ngj - 1)
    def _emit():
        o_ref[...] = _relu(_dot(acc_ref[...], wc_ref[...])
                           + _dot(h_ref[...], ws_ref[...]) + b_ref[...])


def _down_gather_mm(ids2d, table, h, Wc, Ws, b, blk):
    n = h.shape[0]
    return pl.pallas_call(
        _down_body,
        grid=(n // blk, TMAX // SEGB),
        in_specs=[
            pl.BlockSpec((blk, 1), lambda gi, gj: (gi, 0)),
            pl.BlockSpec((SEGB, H), lambda gi, gj: (gj, 0)),
            pl.BlockSpec((blk, H), lambda gi, gj: (gi, 0)),
            pl.BlockSpec((H, H), lambda gi, gj: (0, 0)),
            pl.BlockSpec((H, H), lambda gi, gj: (0, 0)),
            pl.BlockSpec((1, H), lambda gi, gj: (0, 0)),
        ],
        out_specs=pl.BlockSpec((blk, H), lambda gi, gj: (gi, 0)),
        out_shape=jax.ShapeDtypeStruct((n, H), f32),
        scratch_shapes=[pltpu.VMEM((blk, H), f32)],
    )(ids2d, table, h, Wc, Ws, b.reshape(1, H))


def _pool_body(d_ref, id_ref, inv_ref, o_ref):
    i = pl.program_id(0)
    n = pl.num_programs(0)
    oh = (id_ref[...] == lax.broadcasted_iota(i32, (d_ref.shape[0], B), 1)
          ).astype(f32)
    part = lax.dot_general(oh, d_ref[...], (((0,), (0,)), ((), ())),
                           preferred_element_type=f32)

    @pl.when(i == 0)
    def _init():
        o_ref[...] = jnp.zeros_like(o_ref)

    o_ref[...] += part

    @pl.when(i == n - 1)
    def _scale():
        o_ref[...] = o_ref[...] * inv_ref[...]


def _pool_mean(data, ids2d, invb, blk):
    n = data.shape[0]
    return pl.pallas_call(
        _pool_body,
        grid=(n // blk,),
        in_specs=[
            pl.BlockSpec((blk, H), lambda i: (i, 0)),
            pl.BlockSpec((blk, 1), lambda i: (i, 0)),
            pl.BlockSpec((B, H), lambda i: (0, 0)),
        ],
        out_specs=pl.BlockSpec((B, H), lambda i: (0, 0)),
        out_shape=jax.ShapeDtypeStruct((B, H), f32),
    )(data, ids2d, invb)


def _head_body(g0_ref, g1_ref, h2_ref, w1_ref, b1_ref, w2_ref, b2_ref, o_ref):
    hid = _relu(_dot(g0_ref[...], w1_ref[0]) + _dot(g1_ref[...], w1_ref[1])
                + _dot(h2_ref[...], w1_ref[2]) + b1_ref[...])
    o_ref[...] = _dot(hid, w2_ref[...]) + b2_ref[...]


def _head(g0, g1, h2, W1, b1, W2p, b2p):
    return pl.pallas_call(
        _head_body,
        in_specs=[pl.BlockSpec((B, H), lambda: (0, 0))] * 3 + [
            pl.BlockSpec((3, H, H), lambda: (0, 0, 0)),
            pl.BlockSpec((1, H), lambda: (0, 0)),
            pl.BlockSpec((H, H), lambda: (0, 0)),
            pl.BlockSpec((1, H), lambda: (0, 0)),
        ],
        out_specs=pl.BlockSpec((B, H), lambda: (0, 0)),
        out_shape=jax.ShapeDtypeStruct((B, H), f32),
    )(g0, g1, h2, W1.reshape(3, H, H), b1.reshape(1, H), W2p, b2p)


def _gcn_encoder(h, sd4, invdegb, Wself, Wnei, bb):
    for l in range(Wself.shape[0]):
        S = _sc_scatter_sum(h, sd4, NA0)
        h = _fused_layer(h, S, invdegb, Wself[l], Wnei[l], bb[l], 2000)
    return h


def kernel(x, edge_index, batch, assign_index, W_in, b_in, enc_Wself, enc_Wnei,
           enc_b, inter_Wc, inter_Ws, inter_b, head_W1, head_b1, head_W2,
           head_b2):
    src = edge_index[0]
    dst = edge_index[1]

    bids = jnp.arange(B, dtype=i32)
    onehot_b = batch[:, None] == bids[None, :]
    imin = jnp.iinfo(i32).min
    max_func = jnp.max(jnp.where(onehot_b, assign_index[:, None], imin),
                       axis=0)
    nums = max_func + 1
    offsets = jnp.concatenate([jnp.zeros((1,), nums.dtype),
                               jnp.cumsum(nums)[:-1]])
    assign0 = assign_index + offsets[batch]
    bounds = jnp.cumsum(nums)
    assign1 = jnp.sum((jnp.arange(TMAX, dtype=i32)[:, None]
                       >= bounds[None, :]).astype(i32), axis=1)

    sd4 = jnp.concatenate([src.reshape(NW, -1, 1, CH),
                           dst.reshape(NW, -1, 1, CH)], axis=2)

    deg = jax.ops.segment_sum(jnp.ones((E,), f32), dst, num_segments=N)
    invdegb = jnp.broadcast_to((1.0 / jnp.maximum(deg, 1.0))[:, None], (N, H))
    cnt1 = jnp.sum((assign1[:, None] == bids[None, :]).astype(f32), axis=0)
    inv1b = jnp.broadcast_to((1.0 / jnp.maximum(cnt1, 1.0))[:, None], (B, H))
    cntb = jnp.sum(onehot_b.astype(f32), axis=0)
    invbb = jnp.broadcast_to((1.0 / jnp.maximum(cntb, 1.0))[:, None], (B, H))
    batch2d = batch.reshape(N, 1)
    assign1_2d = assign1.reshape(TMAX, 1)
    assign0_2d = assign0.reshape(N, 1)

    x0 = _mm_relu(x, W_in, b_in, 2000)
    x1 = jnp.zeros((TMAX, H), f32)
    x2 = jnp.zeros((B, H), f32)
    h0 = _gcn_encoder(x0, sd4, invdegb, enc_Wself[0], enc_Wnei[0], enc_b[0])
    h1 = _mlp2(x1, enc_Wself[1], enc_b[1], 800)
    h2 = _mlp2(x2, enc_Wself[2], enc_b[2], B)

    for _ in range(2):
        h1n = _up_pool_mm(assign0_2d, h0, h1, inter_Wc[0],
                          inter_Ws[0], inter_b[0], 2000)
        up12 = _pool_mean(h1n, assign1_2d, inv1b, 800)
        h2n = _dual_mm_relu(up12, h2, inter_Wc[1], inter_Ws[1], inter_b[1], B)
        h0 = _down_gather_mm(assign0_2d, h1n, h0, inter_Wc[2], inter_Ws[2],
                             inter_b[2], 2000)
        h1, h2 = h1n, h2n
        h0 = _gcn_encoder(h0, sd4, invdegb, enc_Wself[3], enc_Wnei[3],
                          enc_b[3])
        h1 = _mlp2(h1, enc_Wself[4], enc_b[4], 800)
        h2 = _mlp2(h2, enc_Wself[5], enc_b[5], B)

    g0 = _pool_mean(h0, batch2d, invbb, 2000)
    g1 = _pool_mean(h1, assign1_2d, inv1b, 800)
    W2p = jnp.pad(head_W2, ((0, 0), (0, H - head_W2.shape[1])))
    b2p = jnp.pad(head_b2, (0, H - head_b2.shape[0])).reshape(1, H)
    out = _head(g0, g1, h2, head_W1, head_b1, W2p, b2p)
    return out[:, :head_W2.shape[1]]
